# Initial kernel scaffold; baseline (speedup 1.0000x reference)
import jax
import jax.numpy as jnp
from jax.experimental import pallas as pl


def kernel(x, token_table, position_table):
    raise NotImplementedError("write your pallas kernel here")


# speedup vs baseline: 5.8166x; 5.8166x over previous
import functools

import jax
import jax.numpy as jnp
from jax import lax
from jax.experimental import pallas as pl
from jax.experimental.pallas import tpu as pltpu, tpu_sc as plsc

INPUT_DIM = 100000
D = 32
B = 16384
S = 200

NC = 2
NS = 16
NW = NC * NS
N = B * S
PER_W = N // NW
C = 1600
CHUNKS = PER_W // C


def _embed_body(x_hbm, tok_hbm, pos_hbm, out_hbm, idx_v, rows_v, pos_v, gsem):
    wid = lax.axis_index("s") * NC + lax.axis_index("c")
    base = wid * PER_W

    pltpu.sync_copy(pos_hbm.at[pl.ds(0, S)], pos_v)

    @pl.loop(0, CHUNKS)
    def _chunk(k):
        off = base + k * C
        pltpu.sync_copy(x_hbm.at[pl.ds(off, C)], idx_v)
        pltpu.async_copy(tok_hbm.at[idx_v], rows_v, gsem).wait()

        @pl.loop(0, S)
        def _add(r):
            p0 = pos_v[r, pl.ds(0, 16)]
            p1 = pos_v[r, pl.ds(16, 16)]
            for s in range(C // S):
                plsc.addupdate(rows_v.at[r + s * S, pl.ds(0, 16)], p0)
                plsc.addupdate(rows_v.at[r + s * S, pl.ds(16, 16)], p1)

        pltpu.sync_copy(rows_v, out_hbm.at[pl.ds(off, C)])


@jax.jit
def _embed(x_flat, token_table, position_table):
    mesh = plsc.VectorSubcoreMesh(core_axis_name="c", subcore_axis_name="s")
    return pl.kernel(
        _embed_body,
        out_type=jax.ShapeDtypeStruct((N, D), jnp.float32),
        mesh=mesh,
        compiler_params=pltpu.CompilerParams(use_tc_tiling_on_sc=False),
        scratch_types=[
            pltpu.VMEM((C,), jnp.int32),
            pltpu.VMEM((C, D), jnp.float32),
            pltpu.VMEM((S, D), jnp.float32),
            pltpu.SemaphoreType.DMA,
        ],
    )(x_flat, token_table, position_table)


def kernel(x, token_table, position_table):
    x_flat = x.reshape(-1).astype(jnp.int32)
    out = _embed(x_flat, token_table, position_table)
    return out.reshape(B, S, D)


# speedup vs baseline: 6.3813x; 1.0971x over previous
import functools

import jax
import jax.numpy as jnp
from jax import lax
from jax.experimental import pallas as pl
from jax.experimental.pallas import tpu as pltpu, tpu_sc as plsc

INPUT_DIM = 100000
D = 32
B = 16384
S = 200

NC = 2
NS = 16
NW = NC * NS
N = B * S
PER_W = N // NW
C = 1600
CHUNKS = PER_W // C


def _embed_body(x_hbm, tok_hbm, pos_hbm, out_hbm,
                idx0, idx1, rows0, rows1, pos_v,
                gsem0, gsem1, osem0, osem1, isem0, isem1):
    wid = lax.axis_index("s") * NC + lax.axis_index("c")
    base = wid * PER_W

    pltpu.sync_copy(pos_hbm.at[pl.ds(0, S)], pos_v)

    bufs = ((idx0, rows0, gsem0, osem0, isem0),
            (idx1, rows1, gsem1, osem1, isem1))

    def idx_src(g):
        return x_hbm.at[pl.ds(base + g * C, C)]

    def out_dst(g):
        return out_hbm.at[pl.ds(base + g * C, C)]

    def add_pos(rows_v):
        @pl.loop(0, S)
        def _add(r):
            p0 = pos_v[r, pl.ds(0, 16)]
            p1 = pos_v[r, pl.ds(16, 16)]
            for s in range(C // S):
                plsc.addupdate(rows_v.at[r + s * S, pl.ds(0, 16)], p0)
                plsc.addupdate(rows_v.at[r + s * S, pl.ds(16, 16)], p1)

    pltpu.sync_copy(idx_src(0), idx0)
    pltpu.async_copy(idx_src(1), idx1, isem1)
    pltpu.async_copy(tok_hbm.at[idx0], rows0, gsem0)

    @pl.loop(0, CHUNKS // 2)
    def _pair(k):
        for b in range(2):
            g = 2 * k + b
            c_idx, c_rows, c_gsem, c_osem, c_isem = bufs[b]
            n_idx, n_rows, n_gsem, n_osem, n_isem = bufs[1 - b]

            @pl.when(g + 1 < CHUNKS)
            def _fire_next():
                @pl.when(g >= 1)
                def _wait_prev_scatter():
                    pltpu.make_async_copy(n_rows, out_dst(g - 1), n_osem).wait()
                pltpu.make_async_copy(idx_src(g + 1), n_idx, n_isem).wait()
                pltpu.async_copy(tok_hbm.at[n_idx], n_rows, n_gsem)

            pltpu.make_async_copy(tok_hbm.at[c_idx], c_rows, c_gsem).wait()

            @pl.when(g + 2 < CHUNKS)
            def _prefetch_idx():
                pltpu.async_copy(idx_src(g + 2), c_idx, c_isem)

            add_pos(c_rows)
            pltpu.async_copy(c_rows, out_dst(g), c_osem)

    pltpu.make_async_copy(rows0, out_dst(CHUNKS - 2), osem0).wait()
    pltpu.make_async_copy(rows1, out_dst(CHUNKS - 1), osem1).wait()


@jax.jit
def _embed(x_flat, token_table, position_table):
    mesh = plsc.VectorSubcoreMesh(core_axis_name="c", subcore_axis_name="s")
    return pl.kernel(
        _embed_body,
        out_type=jax.ShapeDtypeStruct((N, D), jnp.float32),
        mesh=mesh,
        compiler_params=pltpu.CompilerParams(use_tc_tiling_on_sc=False),
        scratch_types=[
            pltpu.VMEM((C,), jnp.int32),
            pltpu.VMEM((C,), jnp.int32),
            pltpu.VMEM((C, D), jnp.float32),
            pltpu.VMEM((C, D), jnp.float32),
            pltpu.VMEM((S, D), jnp.float32),
            pltpu.SemaphoreType.DMA,
            pltpu.SemaphoreType.DMA,
            pltpu.SemaphoreType.DMA,
            pltpu.SemaphoreType.DMA,
            pltpu.SemaphoreType.DMA,
            pltpu.SemaphoreType.DMA,
        ],
    )(x_flat, token_table, position_table)


def kernel(x, token_table, position_table):
    x_flat = x.reshape(-1).astype(jnp.int32)
    out = _embed(x_flat, token_table, position_table)
    return out.reshape(B, S, D)
